# Initial kernel scaffold; baseline (speedup 1.0000x reference)
import jax
import jax.numpy as jnp
from jax.experimental import pallas as pl


def kernel(input):
    raise NotImplementedError("write your pallas kernel here")


# speedup vs baseline: 28.0066x; 28.0066x over previous
import functools

import jax
import jax.numpy as jnp
from jax import lax
from jax.experimental import pallas as pl
from jax.experimental.pallas import tpu as pltpu
from jax.experimental.pallas import tpu_sc as plsc

_NB = 5
_L = 16
_NC = 2
_NS = 16
_NW = _NC * _NS
_C2_BIN2 = 1.8944272
_C2_OUT = 0.10557281


def _fpfh_body(inp_hbm, out_hbm, slab, hist, result):
    Bz = inp_hbm.shape[0]
    Pz = inp_hbm.shape[2]
    wid = lax.axis_index("s") * _NC + lax.axis_index("c")
    per_batch = _NW // Bz
    rows = Pz // per_batch
    b = wid // per_batch
    i0 = (wid % per_batch) * rows

    pltpu.sync_copy(inp_hbm.at[b], slab)

    zeros16 = jnp.zeros((_L,), jnp.float32)

    def zero_body(k, c):
        hist[pl.ds(k * _L, _L)] = zeros16
        return c

    lax.fori_loop(0, (_L * 128) // _L, zero_body, 0)

    lane_off = lax.iota(jnp.int32, _L) * 128
    ones16 = jnp.ones((_L,), jnp.float32)

    def j_body(jv, c0):
        js = jv * _L
        pjx = slab[0, pl.ds(js, _L)]
        pjy = slab[1, pl.ds(js, _L)]
        pjz = slab[2, pl.ds(js, _L)]
        njx = slab[3, pl.ds(js, _L)]
        njy = slab[4, pl.ds(js, _L)]
        njz = slab[5, pl.ds(js, _L)]

        def ig_body(ig, c1):
            base = i0 + ig * _L
            pivx = slab[0, pl.ds(base, _L)]
            pivy = slab[1, pl.ds(base, _L)]
            pivz = slab[2, pl.ds(base, _L)]
            nivx = slab[3, pl.ds(base, _L)]
            nivy = slab[4, pl.ds(base, _L)]
            nivz = slab[5, pl.ds(base, _L)]
            for k in range(_L):
                pix = pivx[k]
                piy = pivy[k]
                piz = pivz[k]
                nix = nivx[k]
                niy = nivy[k]
                niz = nivz[k]
                nn = nix * nix + niy * niy + niz * niz

                dx = pjx - pix
                dy = pjy - piy
                dz = pjz - piz
                r2 = dx * dx + dy * dy + dz * dz
                d = nix * dx + niy * dy + niz * dz
                dn = njx * dx + njy * dy + njz * dz
                u = nix * njx + niy * njy + niz * njz
                vx = dy * niz - dz * niy
                vy = dz * nix - dx * niz
                vz = dx * niy - dy * nix
                m = vx * vx + vy * vy + vz * vz
                s = vx * njx + vy * njy + vz * njz
                yv = dn * nn - d * u
                s2 = s * s
                d2 = d * d
                y2 = yv * yv

                b1 = jnp.where(s2 > 0.36 * m, 1.0, 0.0)
                b2 = jnp.where(s2 > 0.04 * m, 1.0, 0.0)
                ia = jnp.where(s >= 0, 2.0 + b1 + b2, 2.0 - b1 - b2)

                q1 = jnp.where(d2 > 0.36 * r2, 1.0, 0.0)
                q2 = jnp.where(d2 > 0.04 * r2, 1.0, 0.0)
                ip = jnp.where(d >= 0, 2.0 + q1 + q2, 2.0 - q1 - q2)

                x2 = u * u * nn * m
                is2 = (u > 0) & (x2 >= _C2_BIN2 * y2)
                iso = (u < 0) & (x2 > _C2_OUT * y2)
                it = jnp.where(
                    is2, 2.0,
                    jnp.where(iso,
                              jnp.where(yv >= 0, 4.0, 0.0),
                              jnp.where(yv >= 0, 3.0, 1.0)))

                fidx = ((ia * 5.0 + ip) * 5.0 + it).astype(jnp.int32) + lane_off
                plsc.addupdate_scatter(hist, [fidx], ones16)
            return c1

        lax.fori_loop(0, rows // _L, ig_body, c0)
        return c0

    lax.fori_loop(0, Pz // _L, j_body, 0)

    def red_body(cch, c):
        acc = zeros16
        for bank in range(_L):
            acc = acc + hist[pl.ds(bank * 128 + cch * _L, _L)]
        result[pl.ds(cch * _L, _L)] = acc
        return c

    lax.fori_loop(0, 128 // _L, red_body, 0)
    pltpu.sync_copy(result, out_hbm.at[wid])


def kernel(input):
    Bz, Pz, _ = input.shape
    inp_t = jnp.transpose(input, (0, 2, 1))

    mesh = plsc.VectorSubcoreMesh(
        core_axis_name="c", subcore_axis_name="s",
        num_cores=_NC, num_subcores=_NS)
    run = functools.partial(
        pl.kernel,
        out_type=jax.ShapeDtypeStruct((_NW, 128), jnp.float32),
        mesh=mesh,
        scratch_types=[
            pltpu.VMEM((6, Pz), jnp.float32),
            pltpu.VMEM((_L * 128,), jnp.float32),
            pltpu.VMEM((128,), jnp.float32),
        ],
        compiler_params=pltpu.CompilerParams(needs_layout_passes=False),
    )(_fpfh_body)
    part = run(inp_t)

    per_batch = _NW // Bz
    hist = part.reshape(Bz, per_batch, 128).sum(axis=1)[:, : _NB ** 3]
    hist = hist.at[:, 62].add(-float(Pz))
    return hist / float(Pz * (Pz - 1))


# speedup vs baseline: 30.3038x; 1.0820x over previous
import functools

import jax
import jax.numpy as jnp
from jax import lax
from jax.experimental import pallas as pl
from jax.experimental.pallas import tpu as pltpu
from jax.experimental.pallas import tpu_sc as plsc

_NB = 5
_L = 16
_NC = 2
_NS = 16
_NW = _NC * _NS
_C2_BIN2 = 1.8944272
_C2_OUT = 0.10557281


def _fpfh_body(inp_hbm, out_hbm, slab, hist, result):
    Bz = inp_hbm.shape[0]
    Pz = inp_hbm.shape[2]
    wid = lax.axis_index("s") * _NC + lax.axis_index("c")
    per_batch = _NW // Bz
    rows = Pz // per_batch
    b = wid // per_batch
    i0 = (wid % per_batch) * rows

    pltpu.sync_copy(inp_hbm.at[b], slab)

    zeros16 = jnp.zeros((_L,), jnp.float32)

    def zero_body(k, c):
        hist[pl.ds(k * _L, _L)] = zeros16
        return c

    lax.fori_loop(0, 128, zero_body, 0)

    lane = lax.iota(jnp.int32, _L)
    base992 = lane.astype(jnp.float32) + float(62 * _L)
    ones16 = jnp.ones((_L,), jnp.float32)

    def j_body(jv, c0):
        js = jv * _L
        pjx = slab[0, pl.ds(js, _L)]
        pjy = slab[1, pl.ds(js, _L)]
        pjz = slab[2, pl.ds(js, _L)]
        njx = slab[3, pl.ds(js, _L)]
        njy = slab[4, pl.ds(js, _L)]
        njz = slab[5, pl.ds(js, _L)]

        def ig_body(ig, c1):
            base = i0 + ig * _L
            pivx = slab[0, pl.ds(base, _L)]
            pivy = slab[1, pl.ds(base, _L)]
            pivz = slab[2, pl.ds(base, _L)]
            nivx = slab[3, pl.ds(base, _L)]
            nivy = slab[4, pl.ds(base, _L)]
            nivz = slab[5, pl.ds(base, _L)]
            for k in range(_L):
                pix = pivx[k]
                piy = pivy[k]
                piz = pivz[k]
                nix = nivx[k]
                niy = nivy[k]
                niz = nivz[k]
                nn = nix * nix + niy * niy + niz * niz

                dx = pjx - pix
                dy = pjy - piy
                dz = pjz - piz
                r2 = dx * dx + dy * dy + dz * dz
                d = nix * dx + niy * dy + niz * dz
                dn = njx * dx + njy * dy + njz * dz
                u = nix * njx + niy * njy + niz * njz
                vx = dy * niz - dz * niy
                vy = dz * nix - dx * niz
                vz = dx * niy - dy * nix
                m = vx * vx + vy * vy + vz * vz
                s = vx * njx + vy * njy + vz * njz
                yv = dn * nn - d * u
                s2 = s * s
                d2 = d * d
                y2 = yv * yv

                ca = (jnp.where(s2 > 0.36 * m, 400.0, 0.0)
                      + jnp.where(s2 > 0.04 * m, 400.0, 0.0))
                sa = jnp.where(s >= 0, 1.0, -1.0)
                cp = (jnp.where(d2 > 0.36 * r2, 80.0, 0.0)
                      + jnp.where(d2 > 0.04 * r2, 80.0, 0.0))
                sp = jnp.where(d >= 0, 1.0, -1.0)
                x2 = u * u * nn * m
                is2 = (u > 0) & (x2 >= _C2_BIN2 * y2)
                iso = (u < 0) & (x2 > _C2_OUT * y2)
                sy = jnp.where(yv >= 0, 16.0, -16.0)
                dt = jnp.where(is2, 0.0, sy + jnp.where(iso, sy, 0.0))

                addr_f = base992 + sa * ca + sp * cp + dt
                addr = addr_f.astype(jnp.int32)
                plsc.addupdate_scatter(hist, [addr], ones16)
            return c1

        lax.fori_loop(0, rows // _L, ig_body, c0)
        return c0

    lax.fori_loop(0, Pz // _L, j_body, 0)

    def red_body(cch, c):
        addr0 = lane * _L + cch * (_L * _L)
        acc = zeros16
        for l in range(_L):
            acc = acc + plsc.load_gather(hist, [addr0 + l])
        result[pl.ds(cch * _L, _L)] = acc
        return c

    lax.fori_loop(0, 128 // _L, red_body, 0)
    pltpu.sync_copy(result, out_hbm.at[wid])


def kernel(input):
    Bz, Pz, _ = input.shape
    inp_t = jnp.transpose(input, (0, 2, 1))

    mesh = plsc.VectorSubcoreMesh(
        core_axis_name="c", subcore_axis_name="s",
        num_cores=_NC, num_subcores=_NS)
    run = functools.partial(
        pl.kernel,
        out_type=jax.ShapeDtypeStruct((_NW, 128), jnp.float32),
        mesh=mesh,
        scratch_types=[
            pltpu.VMEM((6, Pz), jnp.float32),
            pltpu.VMEM((128 * _L,), jnp.float32),
            pltpu.VMEM((128,), jnp.float32),
        ],
        compiler_params=pltpu.CompilerParams(needs_layout_passes=False),
    )(_fpfh_body)
    part = run(inp_t)

    per_batch = _NW // Bz
    hist = part.reshape(Bz, per_batch, 128).sum(axis=1)[:, : _NB ** 3]
    hist = hist.at[:, 62].add(-float(Pz))
    return hist / float(Pz * (Pz - 1))


# speedup vs baseline: 39.7796x; 1.3127x over previous
import functools

import jax
import jax.numpy as jnp
from jax import lax
from jax.experimental import pallas as pl
from jax.experimental.pallas import tpu as pltpu
from jax.experimental.pallas import tpu_sc as plsc

_NB = 5
_L = 16
_NC = 2
_NS = 16
_NW = _NC * _NS
_C2_BIN2 = 1.8944272
_C2_OUT = 0.10557281


def _fpfh_body(inp_hbm, out_hbm, slab, hist, result):
    Bz = inp_hbm.shape[0]
    Pz = inp_hbm.shape[2]
    wid = lax.axis_index("s") * _NC + lax.axis_index("c")
    per_batch = _NW // Bz
    half = Pz // (2 * per_batch)
    grp = half // _L
    nchunk = Pz // _L
    b = wid // per_batch
    q = wid % per_batch

    pltpu.sync_copy(inp_hbm.at[b], slab)

    zeros16 = jnp.zeros((_L,), jnp.float32)

    def zero_body(k, c):
        hist[pl.ds(k * _L, _L)] = zeros16
        return c

    lax.fori_loop(0, 128, zero_body, 0)

    lane = lax.iota(jnp.int32, _L)
    base992 = lane.astype(jnp.float32) + float(62 * _L)
    ones16 = jnp.ones((_L,), jnp.float32)

    for blk in range(2):
        base_blk = q * half if blk == 0 else (Pz - half) - q * half

        def g_body(g, c0, base_blk=base_blk):
            gb = base_blk + g * _L
            jc0 = gb // _L
            pivx = slab[0, pl.ds(gb, _L)]
            pivy = slab[1, pl.ds(gb, _L)]
            pivz = slab[2, pl.ds(gb, _L)]
            nivx = slab[3, pl.ds(gb, _L)]
            nivy = slab[4, pl.ds(gb, _L)]
            nivz = slab[5, pl.ds(gb, _L)]

            def jc_body(jc, c1):
                js = jc * _L
                pjx = slab[0, pl.ds(js, _L)]
                pjy = slab[1, pl.ds(js, _L)]
                pjz = slab[2, pl.ds(js, _L)]
                njx = slab[3, pl.ds(js, _L)]
                njy = slab[4, pl.ds(js, _L)]
                njz = slab[5, pl.ds(js, _L)]
                nnj = njx * njx + njy * njy + njz * njz
                jglob = lane + js
                for k in range(_L):
                    pix = pivx[k]
                    piy = pivy[k]
                    piz = pivz[k]
                    nix = nivx[k]
                    niy = nivy[k]
                    niz = nivz[k]
                    nn = nix * nix + niy * niy + niz * niz
                    mask = jglob > (gb + k)

                    dx = pjx - pix
                    dy = pjy - piy
                    dz = pjz - piz
                    r2 = dx * dx + dy * dy + dz * dz
                    d = nix * dx + niy * dy + niz * dz
                    dn = njx * dx + njy * dy + njz * dz
                    u = nix * njx + niy * njy + niz * njz
                    vx = dy * niz - dz * niy
                    vy = dz * nix - dx * niz
                    vz = dx * niy - dy * nix
                    m = vx * vx + vy * vy + vz * vz
                    s = vx * njx + vy * njy + vz * njz
                    yv = dn * nn - d * u
                    s2 = s * s
                    d2 = d * d
                    y2 = yv * yv
                    uu = u * u
                    sa = jnp.where(s >= 0, 1.0, -1.0)
                    tpa = 0.36 * r2
                    tpb = 0.04 * r2

                    ca = (jnp.where(s2 > 0.36 * m, 400.0, 0.0)
                          + jnp.where(s2 > 0.04 * m, 400.0, 0.0))
                    cp = (jnp.where(d2 > tpa, 80.0, 0.0)
                          + jnp.where(d2 > tpb, 80.0, 0.0))
                    sp = jnp.where(d >= 0, 1.0, -1.0)
                    x2 = uu * nn * m
                    is2 = (u > 0) & (x2 >= _C2_BIN2 * y2)
                    iso = (u < 0) & (x2 > _C2_OUT * y2)
                    sy = jnp.where(yv >= 0, 16.0, -16.0)
                    dt = jnp.where(is2, 0.0, sy + jnp.where(iso, sy, 0.0))
                    addr = (base992 + sa * ca + sp * cp + dt).astype(jnp.int32)
                    plsc.addupdate_scatter(hist, [addr], ones16, mask=mask)

                    ddn = dn * dn
                    mr = r2 * nnj - ddn
                    yr = dn * u - d * nnj
                    y2r = yr * yr
                    car = (jnp.where(s2 > 0.36 * mr, 400.0, 0.0)
                           + jnp.where(s2 > 0.04 * mr, 400.0, 0.0))
                    cpr = (jnp.where(ddn > tpa, 80.0, 0.0)
                           + jnp.where(ddn > tpb, 80.0, 0.0))
                    spr = jnp.where(dn <= 0, 1.0, -1.0)
                    x2r = uu * nnj * mr
                    is2r = (u > 0) & (x2r >= _C2_BIN2 * y2r)
                    isor = (u < 0) & (x2r > _C2_OUT * y2r)
                    syr = jnp.where(yr >= 0, 16.0, -16.0)
                    dtr = jnp.where(is2r, 0.0, syr + jnp.where(isor, syr, 0.0))
                    addr_r = (base992 + sa * car + spr * cpr + dtr).astype(jnp.int32)
                    plsc.addupdate_scatter(hist, [addr_r], ones16, mask=mask)
                return c1

            lax.fori_loop(jc0, nchunk, jc_body, c0)
            return c0

        lax.fori_loop(0, grp, g_body, 0)

    def red_body(cch, c):
        addr0 = lane * _L + cch * (_L * _L)
        acc = zeros16
        for l in range(_L):
            acc = acc + plsc.load_gather(hist, [addr0 + l])
        result[pl.ds(cch * _L, _L)] = acc
        return c

    lax.fori_loop(0, 128 // _L, red_body, 0)
    pltpu.sync_copy(result, out_hbm.at[wid])


def kernel(input):
    Bz, Pz, _ = input.shape
    inp_t = jnp.transpose(input, (0, 2, 1))

    mesh = plsc.VectorSubcoreMesh(
        core_axis_name="c", subcore_axis_name="s",
        num_cores=_NC, num_subcores=_NS)
    run = functools.partial(
        pl.kernel,
        out_type=jax.ShapeDtypeStruct((_NW, 128), jnp.float32),
        mesh=mesh,
        scratch_types=[
            pltpu.VMEM((6, Pz), jnp.float32),
            pltpu.VMEM((128 * _L,), jnp.float32),
            pltpu.VMEM((128,), jnp.float32),
        ],
        compiler_params=pltpu.CompilerParams(needs_layout_passes=False),
    )(_fpfh_body)
    part = run(inp_t)

    per_batch = _NW // Bz
    hist = part.reshape(Bz, per_batch, 128).sum(axis=1)[:, : _NB ** 3]
    return hist / float(Pz * (Pz - 1))
